# baseline (speedup 1.0000x reference)
import functools

import jax
import jax.numpy as jnp
from jax import lax
from jax.experimental import pallas as pl
from jax.experimental.pallas import tpu as pltpu
from jax.experimental.pallas import tpu_sc as plsc

N = 20000
CONF = 0.5
NC = 1
NS = 8
L = 16
NW = NC * NS
CHUNK = 2496
LAST = N - (NW - 1) * CHUNK
NITER = CHUNK // L
NITER_LAST = LAST // L
BUF = max(CHUNK, LAST)

_mesh = plsc.VectorSubcoreMesh(
    core_axis_name="c", subcore_axis_name="s", num_cores=NC, num_subcores=NS
)


@functools.partial(
    pl.kernel,
    out_type=jax.ShapeDtypeStruct((NW, L), jnp.float32),
    mesh=_mesh,
    compiler_params=pltpu.CompilerParams(
        needs_layout_passes=False,
        skip_device_barrier=True,
    ),
    scratch_types=[
        pltpu.VMEM((BUF,), jnp.float32),
        pltpu.VMEM((BUF,), jnp.float32),
        pltpu.VMEM((L,), jnp.float32),
        pltpu.SemaphoreType.DMA,
    ],
)
def _masked_sum_sc(cls_hbm, eps_hbm, out_hbm, cls_v, eps_v, acc_v, sem):
    wid = lax.axis_index("s") * NC + lax.axis_index("c")
    base = wid * CHUNK
    is_last = wid == NW - 1

    @pl.when(jnp.logical_not(is_last))
    def _():
        c1 = pltpu.async_copy(
            cls_hbm.at[pl.ds(base, CHUNK)], cls_v.at[pl.ds(0, CHUNK)], sem)
        c2 = pltpu.async_copy(
            eps_hbm.at[pl.ds(base, CHUNK)], eps_v.at[pl.ds(0, CHUNK)], sem)
        c1.wait()
        c2.wait()

    @pl.when(is_last)
    def _():
        c1 = pltpu.async_copy(
            cls_hbm.at[pl.ds(base, LAST)], cls_v.at[pl.ds(0, LAST)], sem)
        c2 = pltpu.async_copy(
            eps_hbm.at[pl.ds(base, LAST)], eps_v.at[pl.ds(0, LAST)], sem)
        c1.wait()
        c2.wait()

    lanes = lax.iota(jnp.int32, L)
    my_len = jnp.where(is_last, LAST, CHUNK)

    def body(k, acc):
        off = k * L
        c = cls_v[pl.ds(off, L)]
        s = eps_v[pl.ds(off, L)]
        valid = (off + lanes < my_len) & (c >= CONF)
        return acc + jnp.where(valid, c + s, 0.0)

    acc = lax.fori_loop(0, NITER_LAST, body, jnp.zeros((L,), jnp.float32))
    acc_v[...] = acc
    pltpu.sync_copy(acc_v, out_hbm.at[wid])


def kernel(cls_scores, end_points):
    partials = _masked_sum_sc(cls_scores, end_points.sum(axis=1))
    return jnp.sum(partials)


# speedup vs baseline: 1.0197x; 1.0197x over previous
import functools

import jax
import jax.numpy as jnp
from jax import lax
from jax.experimental import pallas as pl
from jax.experimental.pallas import tpu as pltpu
from jax.experimental.pallas import tpu_sc as plsc

N = 20000
CONF = 0.5
NC = 1
NS = 16
L = 16
NW = NC * NS
CHUNK = 1248
LAST = N - (NW - 1) * CHUNK
NITER = CHUNK // L
NITER_LAST = LAST // L
BUF = max(CHUNK, LAST)

_mesh = plsc.VectorSubcoreMesh(
    core_axis_name="c", subcore_axis_name="s", num_cores=NC, num_subcores=NS
)


@functools.partial(
    pl.kernel,
    out_type=jax.ShapeDtypeStruct((NW, L), jnp.float32),
    mesh=_mesh,
    compiler_params=pltpu.CompilerParams(
        needs_layout_passes=False,
        skip_device_barrier=True,
    ),
    scratch_types=[
        pltpu.VMEM((BUF,), jnp.float32),
        pltpu.VMEM((BUF,), jnp.float32),
        pltpu.VMEM((L,), jnp.float32),
        pltpu.SemaphoreType.DMA,
    ],
)
def _masked_sum_sc(cls_hbm, eps_hbm, out_hbm, cls_v, eps_v, acc_v, sem):
    wid = lax.axis_index("s") * NC + lax.axis_index("c")
    base = wid * CHUNK
    is_last = wid == NW - 1

    @pl.when(jnp.logical_not(is_last))
    def _():
        c1 = pltpu.async_copy(
            cls_hbm.at[pl.ds(base, CHUNK)], cls_v.at[pl.ds(0, CHUNK)], sem)
        c2 = pltpu.async_copy(
            eps_hbm.at[pl.ds(base, CHUNK)], eps_v.at[pl.ds(0, CHUNK)], sem)
        c1.wait()
        c2.wait()

    @pl.when(is_last)
    def _():
        c1 = pltpu.async_copy(
            cls_hbm.at[pl.ds(base, LAST)], cls_v.at[pl.ds(0, LAST)], sem)
        c2 = pltpu.async_copy(
            eps_hbm.at[pl.ds(base, LAST)], eps_v.at[pl.ds(0, LAST)], sem)
        c1.wait()
        c2.wait()

    lanes = lax.iota(jnp.int32, L)
    my_len = jnp.where(is_last, LAST, CHUNK)

    def body(k, acc):
        off = k * L
        c = cls_v[pl.ds(off, L)]
        s = eps_v[pl.ds(off, L)]
        valid = (off + lanes < my_len) & (c >= CONF)
        return acc + jnp.where(valid, c + s, 0.0)

    acc = lax.fori_loop(0, NITER_LAST, body, jnp.zeros((L,), jnp.float32))
    acc_v[...] = acc
    pltpu.sync_copy(acc_v, out_hbm.at[wid])


def kernel(cls_scores, end_points):
    partials = _masked_sum_sc(cls_scores, end_points.sum(axis=1))
    return jnp.sum(partials)
